# baseline (speedup 1.0000x reference)
import functools

import jax
import jax.numpy as jnp
from jax import lax
from jax.experimental import pallas as pl
from jax.experimental.pallas import tpu as pltpu
from jax.experimental.pallas import tpu_sc as plsc

B = 16384
HID = 64
CD = 3 * HID
PAD = 128
NCT = 100

NPERT = 100001
_TB = 2048
_NBLK = 25
NH = _NBLK * _TB
_QROWS = NH

_NC = 2
_NS = 16
_NW = _NC * _NS
_BPW = B // _NW
_CHUNK = 128
_NCH = _BPW // _CHUNK


def _tp_body(xl_ref, xr_ref, eye_ref, o_ref):
    i = pl.program_id(0)
    xl = xl_ref[...]
    xr = xr_ref[...]
    cols = NH + i * _TB + lax.broadcasted_iota(jnp.int32, (HID, _TB), 1)
    xr = jnp.where(cols < NPERT, xr, 0.0)
    _cn = (((0,), (0,)), ((), ()))
    xlt = lax.dot_general(xl, eye_ref[...], _cn,
                          preferred_element_type=jnp.float32)
    xrt = xr.T
    o_ref[...] = jnp.concatenate([xlt, xrt], axis=1)


def _transpose_pack(pt_t, eye):
    return pl.pallas_call(
        _tp_body,
        grid=(_NBLK,),
        in_specs=[
            pl.BlockSpec((HID, _TB), lambda i: (0, i)),
            pl.BlockSpec((HID, _TB), lambda i: (0, jnp.minimum(i + _NBLK, (NPERT - 1) // _TB))),
            pl.BlockSpec((HID, HID), lambda i: (0, 0)),
        ],
        out_specs=pl.BlockSpec((_TB, PAD), lambda i: (i, 0)),
        out_shape=jax.ShapeDtypeStruct((_QROWS, PAD), jnp.float32),
    )(pt_t, pt_t, eye)


def _sc_gather_body(tab, idx, out, idx_v, st0, st1, gsem, osem):
    wid = lax.axis_index("s") * _NC + lax.axis_index("c")
    base = wid * _BPW
    pltpu.sync_copy(idx.at[pl.ds(base, _BPW)], idx_v)
    st = (st0, st1)
    gcp = [None] * _NCH
    ocp = [None] * _NCH
    gcp[0] = pltpu.async_copy(tab.at[idx_v.at[pl.ds(0, _CHUNK)]], st0, gsem)
    for j in range(_NCH):
        if j + 1 < _NCH:
            if j >= 1:
                ocp[j - 1].wait()
            gcp[j + 1] = pltpu.async_copy(
                tab.at[idx_v.at[pl.ds((j + 1) * _CHUNK, _CHUNK)]],
                st[(j + 1) % 2], gsem)
        gcp[j].wait()
        ocp[j] = pltpu.async_copy(
            st[j % 2], out.at[pl.ds(base + j * _CHUNK, _CHUNK), :], osem)
    ocp[_NCH - 2].wait()
    ocp[_NCH - 1].wait()


def _make_sc_gather():
    return functools.partial(
        pl.kernel,
        mesh=plsc.VectorSubcoreMesh(core_axis_name="c", subcore_axis_name="s"),
        out_type=jax.ShapeDtypeStruct((B, PAD), jnp.float32),
        scratch_types=[
            pltpu.VMEM((_BPW,), jnp.int32),
            pltpu.VMEM((_CHUNK, PAD), jnp.float32),
            pltpu.VMEM((_CHUNK, PAD), jnp.float32),
            pltpu.SemaphoreType.DMA,
            pltpu.SemaphoreType.DMA,
        ],
    )(_sc_gather_body)


_sc_gather_p = _make_sc_gather()
_sc_gather_b = _make_sc_gather()


_MB = 1024
_DN = (((0,), (1,)), ((), ()))
_DT = (((0,), (0,)), ((), ()))
_DN_P2 = (((1,), (0,)), ((), ()))


def _mm_body(x1, x3, pr, ci, ctt, wa, wd, w2, w3, bb, o):
    acc = lax.dot_general(wa[...], x1[...], _DN, preferred_element_type=jnp.float32)
    acc += pr[...] * lax.dot_general(wd[...], x1[...], _DN,
                                     preferred_element_type=jnp.float32)
    acc += lax.dot_general(w3[...], x3[...], _DN, preferred_element_type=jnp.float32)
    p2t = lax.dot_general(w2[...], ctt[...], _DT,
                          preferred_element_type=jnp.float32)
    rows = lax.broadcasted_iota(jnp.int32, (NCT, _MB), 0)
    oh = (rows == ci[...]).astype(jnp.float32)
    acc += lax.dot_general(p2t, oh, _DN_P2, preferred_element_type=jnp.float32)
    o[...] = acc + bb[...]


def _matmul_t(pe, be, pr, ci, ctt, wa, wd, w2, w3, bcol):
    return pl.pallas_call(
        _mm_body,
        grid=(B // _MB,),
        in_specs=[
            pl.BlockSpec((_MB, PAD), lambda i: (i, 0)),
            pl.BlockSpec((_MB, PAD), lambda i: (i, 0)),
            pl.BlockSpec((1, _MB), lambda i: (0, i)),
            pl.BlockSpec((1, _MB), lambda i: (0, i)),
            pl.BlockSpec((HID, NCT), lambda i: (0, 0)),
            pl.BlockSpec((PAD, CD), lambda i: (0, 0)),
            pl.BlockSpec((PAD, CD), lambda i: (0, 0)),
            pl.BlockSpec((HID, CD), lambda i: (0, 0)),
            pl.BlockSpec((PAD, CD), lambda i: (0, 0)),
            pl.BlockSpec((CD, 1), lambda i: (0, 0)),
        ],
        out_specs=pl.BlockSpec((CD, _MB), lambda i: (0, i)),
        out_shape=jax.ShapeDtypeStruct((CD, B), jnp.float32),
    )(pe, be, pr, ci, ctt, wa, wd, w2, w3, bcol)


def kernel(pert_table, celltype_table, batch_table, W_gather, b_gather,
           pert_idx, celltype_idx, batch_idx):
    pidx = pert_idx.astype(jnp.int32)
    cidx = celltype_idx.astype(jnp.int32)
    bidx = batch_idx.astype(jnp.int32)
    eye = jnp.eye(HID, dtype=jnp.float32)
    btab = jnp.pad(batch_table, ((0, 0), (0, PAD - HID)))
    be = _sc_gather_b(btab, bidx)
    ptab = _transpose_pack(pert_table.T, eye)
    right = pidx >= NH
    pidx_q = jnp.where(right, pidx - NH, pidx)
    parity = right.astype(jnp.float32).reshape(1, B)
    ci = cidx.reshape(1, B)
    pe = _sc_gather_p(ptab, pidx_q)
    zpad = jnp.zeros((PAD - HID, CD), jnp.float32)
    w1 = W_gather[:HID, :]
    wa = jnp.concatenate([w1, zpad], axis=0)
    wd = jnp.concatenate([-w1, w1], axis=0)
    w2 = W_gather[HID:2 * HID, :]
    w3 = jnp.concatenate([W_gather[2 * HID:, :], zpad], axis=0)
    bcol = b_gather.reshape(CD, 1)
    out_t = _matmul_t(pe, be, parity, ci, celltype_table.T, wa, wd, w2, w3, bcol)
    return out_t.T
